# baseline (speedup 1.0000x reference)
import functools

import jax
import jax.numpy as jnp
from jax import lax
from jax.experimental import pallas as pl
from jax.experimental.pallas import tpu as pltpu
from jax.experimental.pallas import tpu_sc as plsc

_BATCH = 16384
_FEATURES = 512
_PATHWAYS = 64
_GROUP = 8

_NUM_CORES = 2
_NUM_SUBCORES = 16
_NUM_WORKERS = _NUM_CORES * _NUM_SUBCORES
_ROWS_PER_WORKER = _BATCH // _NUM_WORKERS
_CHUNK = 64
_NUM_CHUNKS = _ROWS_PER_WORKER // _CHUNK
_LANES = 16
_QVECS = _PATHWAYS // _LANES


def _body(x_hbm, out_hbm, in_v, out_v, sem_in):
    wid = lax.axis_index("s") * _NUM_CORES + lax.axis_index("c")
    row0 = wid * _ROWS_PER_WORKER

    lane = lax.iota(jnp.int32, _LANES)
    col_idx = [
        [lane * _GROUP + (q * _LANES * _GROUP + k) for k in range(_GROUP)]
        for q in range(_QVECS)
    ]

    def do_chunk(c, _):
        rowa = row0 + c * _CHUNK
        pltpu.sync_copy(
            x_hbm.at[pl.ds(rowa * _FEATURES, _CHUNK * _FEATURES)], in_v
        )

        def do_row(r, _):
            rbase = jnp.full((_LANES,), r * _FEATURES, dtype=jnp.int32)
            for q in range(_QVECS):
                acc = plsc.load_gather(in_v, [rbase + col_idx[q][0]])
                for k in range(1, _GROUP):
                    acc = acc + plsc.load_gather(in_v, [rbase + col_idx[q][k]])
                out_v[pl.ds(r * _PATHWAYS + q * _LANES, _LANES)] = acc * (
                    1.0 / _GROUP
                )
            return 0

        lax.fori_loop(0, _CHUNK, do_row, 0)
        pltpu.sync_copy(
            out_v, out_hbm.at[pl.ds(rowa * _PATHWAYS, _CHUNK * _PATHWAYS)]
        )
        return 0

    lax.fori_loop(0, _NUM_CHUNKS, do_chunk, 0)


@jax.jit
def kernel(gene_set_features):
    mesh = plsc.VectorSubcoreMesh(core_axis_name="c", subcore_axis_name="s")
    run = functools.partial(
        pl.kernel,
        out_type=jax.ShapeDtypeStruct((_BATCH * _PATHWAYS,), jnp.float32),
        mesh=mesh,
        scratch_types=[
            pltpu.VMEM((_CHUNK * _FEATURES,), jnp.float32),
            pltpu.VMEM((_CHUNK * _PATHWAYS,), jnp.float32),
            pltpu.SemaphoreType.DMA,
        ],
        compiler_params=pltpu.CompilerParams(needs_layout_passes=False),
    )(_body)
    flat = run(gene_set_features.reshape(-1))
    return flat.reshape(_BATCH, _PATHWAYS)


# speedup vs baseline: 1.2078x; 1.2078x over previous
import functools

import jax
import jax.numpy as jnp
from jax import lax
from jax.experimental import pallas as pl
from jax.experimental.pallas import tpu as pltpu
from jax.experimental.pallas import tpu_sc as plsc

_BATCH = 16384
_FEATURES = 512
_PATHWAYS = 64
_GROUP = 8

_NUM_CORES = 2
_NUM_SUBCORES = 16
_NUM_WORKERS = _NUM_CORES * _NUM_SUBCORES
_ROWS_PER_WORKER = _BATCH // _NUM_WORKERS
_CHUNK = 64
_NUM_CHUNKS = _ROWS_PER_WORKER // _CHUNK
_LANES = 16
_QVECS = _PATHWAYS // _LANES


def _body(x_hbm, out_hbm, in_v0, in_v1, out_v0, out_v1, si0, si1, so0, so1):
    wid = lax.axis_index("s") * _NUM_CORES + lax.axis_index("c")
    row0 = wid * _ROWS_PER_WORKER

    in_bufs = [in_v0, in_v1]
    out_bufs = [out_v0, out_v1]
    in_sems = [si0, si1]
    out_sems = [so0, so1]

    lane = lax.iota(jnp.int32, _LANES)
    col_idx = [
        [lane * _GROUP + (q * _LANES * _GROUP + k) for k in range(_GROUP)]
        for q in range(_QVECS)
    ]

    def start_in(c):
        rowa = row0 + c * _CHUNK
        return pltpu.async_copy(
            x_hbm.at[pl.ds(rowa * _FEATURES, _CHUNK * _FEATURES)],
            in_bufs[c & 1],
            in_sems[c & 1],
        )

    in_descs = [start_in(0), None]
    out_descs = [None, None]
    for c in range(_NUM_CHUNKS):
        b = c & 1
        if c + 1 < _NUM_CHUNKS:
            in_descs[1 - b] = start_in(c + 1)
        in_descs[b].wait()
        if out_descs[b] is not None:
            out_descs[b].wait()
        in_v = in_bufs[b]
        out_v = out_bufs[b]

        @plsc.parallel_loop(0, _CHUNK, 1, unroll=2)
        def do_row(r):
            rbase = jnp.full((_LANES,), r * _FEATURES, dtype=jnp.int32)
            for q in range(_QVECS):
                acc = plsc.load_gather(in_v, [rbase + col_idx[q][0]])
                for k in range(1, _GROUP):
                    acc = acc + plsc.load_gather(in_v, [rbase + col_idx[q][k]])
                out_v[pl.ds(r * _PATHWAYS + q * _LANES, _LANES)] = acc * (
                    1.0 / _GROUP
                )

        rowa = row0 + c * _CHUNK
        out_descs[b] = pltpu.async_copy(
            out_v,
            out_hbm.at[pl.ds(rowa * _PATHWAYS, _CHUNK * _PATHWAYS)],
            out_sems[b],
        )
    out_descs[0].wait()
    out_descs[1].wait()


@jax.jit
def kernel(gene_set_features):
    mesh = plsc.VectorSubcoreMesh(core_axis_name="c", subcore_axis_name="s")
    run = functools.partial(
        pl.kernel,
        out_type=jax.ShapeDtypeStruct((_BATCH * _PATHWAYS,), jnp.float32),
        mesh=mesh,
        scratch_types=[
            pltpu.VMEM((_CHUNK * _FEATURES,), jnp.float32),
            pltpu.VMEM((_CHUNK * _FEATURES,), jnp.float32),
            pltpu.VMEM((_CHUNK * _PATHWAYS,), jnp.float32),
            pltpu.VMEM((_CHUNK * _PATHWAYS,), jnp.float32),
            pltpu.SemaphoreType.DMA,
            pltpu.SemaphoreType.DMA,
            pltpu.SemaphoreType.DMA,
            pltpu.SemaphoreType.DMA,
        ],
        compiler_params=pltpu.CompilerParams(needs_layout_passes=False),
    )(_body)
    flat = run(gene_set_features.reshape(-1))
    return flat.reshape(_BATCH, _PATHWAYS)
